# baseline (speedup 1.0000x reference)
import jax
import jax.numpy as jnp
from jax import lax
from jax.experimental import pallas as pl
from jax.experimental.pallas import tpu as pltpu
from jax.experimental.pallas import tpu_sc as plsc

B = 16384
D = 64
HIST = 50
NC, NS = 2, 16
NW = NC * NS
BPW = B // NW
RPC = 2 * HIST
CHUNKS = BPW // 2
NBUF = 2
LROW = 128
LABROWS = BPW // LROW


def _accum(rows_v, c, acc_v):
    for b in range(2):
        for d in range(4):
            t = rows_v[b * HIST, pl.ds(d * 16, 16)]
            for j in range(1, HIST):
                t = t + rows_v[b * HIST + j, pl.ds(d * 16, 16)]
            acc_v[2 * c + b, pl.ds(d * 16, 16)] = t


def _sc_body(idx_hbm, lab_hbm, symp_tab, dise_tab, sum_out, dise_out,
             idx_v, lab_v, rows_v, db0_v, db1_v, acc_v,
             sem0, sem1, dsem0, dsem1):
    wid = lax.axis_index("s") * NC + lax.axis_index("c")
    pltpu.sync_copy(lab_hbm.at[pl.ds(wid * LABROWS, LABROWS)], lab_v)
    pltpu.async_copy(dise_tab.at[lab_v.at[0]], db0_v, dsem0)
    pltpu.async_copy(dise_tab.at[lab_v.at[1]], db1_v, dsem1)
    pltpu.sync_copy(idx_hbm.at[pl.ds(wid * CHUNKS, CHUNKS)], idx_v)
    for r in range(LABROWS):
        buf = db0_v if r % 2 == 0 else db1_v
        sem = dsem0 if r % 2 == 0 else dsem1
        pltpu.make_async_copy(dise_tab.at[lab_v.at[r]], buf, sem).wait()
        pltpu.sync_copy(buf, dise_out.at[pl.ds(wid * BPW + r * LROW, LROW)])
        if r + 2 < LABROWS:
            pltpu.async_copy(dise_tab.at[lab_v.at[r + 2]], buf, sem)

    bufsems = (sem0, sem1)

    def gather(c, k):
        return pltpu.async_copy(
            symp_tab.at[idx_v.at[c]], rows_v.at[k], bufsems[k])

    def wait(c, k):
        pltpu.make_async_copy(
            symp_tab.at[idx_v.at[c]], rows_v.at[k], bufsems[k]).wait()

    for k in range(NBUF):
        gather(k, k)

    def quad_body(q, carry):
        c0 = q * NBUF
        for k in range(NBUF):
            wait(c0 + k, k)
            _accum(rows_v.at[k], c0 + k, acc_v)

            @pl.when(q < CHUNKS // NBUF - 1)
            def _():
                gather(c0 + k + NBUF, k)

        return carry

    lax.fori_loop(0, CHUNKS // NBUF, quad_body, 0)
    pltpu.sync_copy(acc_v, sum_out.at[pl.ds(wid * BPW, BPW)])


_sc_gather = pl.kernel(
    _sc_body,
    out_type=(jax.ShapeDtypeStruct((B, D), jnp.float32),
              jax.ShapeDtypeStruct((B, D), jnp.float32)),
    mesh=plsc.VectorSubcoreMesh(core_axis_name="c", subcore_axis_name="s"),
    scratch_types=[
        pltpu.VMEM((CHUNKS, RPC), jnp.int32),
        pltpu.VMEM((LABROWS, LROW), jnp.int32),
        pltpu.VMEM((NBUF, RPC, D), jnp.float32),
        pltpu.VMEM((LROW, D), jnp.float32),
        pltpu.VMEM((LROW, D), jnp.float32),
        pltpu.VMEM((BPW, D), jnp.float32),
        pltpu.SemaphoreType.DMA,
        pltpu.SemaphoreType.DMA,
        pltpu.SemaphoreType.DMA,
        pltpu.SemaphoreType.DMA,
    ],
    compiler_params=pltpu.CompilerParams(use_tc_tiling_on_sc=False),
)


BLK = 512


def _mlp_body(sum_ref, dise_ref, symp_ref, w1_ref, b1_ref, w2t_ref, b2_ref,
              out_ref):
    s = symp_ref[...]
    cnt = jnp.sum((s != 0).astype(jnp.float32), axis=1, keepdims=True)
    w = 1.0 / (cnt + 1e-8)
    w = jnp.where(w >= 1e8, 0.0, w)
    u = jnp.maximum(sum_ref[...] * w, 0.0)
    dd = jnp.maximum(dise_ref[...], 0.0)
    w1 = w1_ref[...]
    h = (jnp.dot(u, w1[:D], preferred_element_type=jnp.float32)
         + jnp.dot(dd, w1[D:], preferred_element_type=jnp.float32)
         + b1_ref[...])
    h = jnp.maximum(h, 0.0)
    out_ref[...] = (jnp.sum(h * w2t_ref[...], axis=1, keepdims=True)
                    + b2_ref[...])


def _mlp(emb_sum, emb_dise, symp, W1, b1r, W2t, b2r):
    hist = symp.shape[1]
    return pl.pallas_call(
        _mlp_body,
        grid=(B // BLK,),
        in_specs=[
            pl.BlockSpec((BLK, D), lambda i: (i, 0)),
            pl.BlockSpec((BLK, D), lambda i: (i, 0)),
            pl.BlockSpec((BLK, hist), lambda i: (i, 0)),
            pl.BlockSpec((2 * D, D), lambda i: (0, 0)),
            pl.BlockSpec((1, D), lambda i: (0, 0)),
            pl.BlockSpec((1, D), lambda i: (0, 0)),
            pl.BlockSpec((1, 1), lambda i: (0, 0)),
        ],
        out_specs=pl.BlockSpec((BLK, 1), lambda i: (i, 0)),
        out_shape=jax.ShapeDtypeStruct((B, 1), jnp.float32),
    )(emb_sum, emb_dise, symp, W1, b1r, W2t, b2r)


def kernel(symp, label, symp_table, dise_table, W1, b1, W2, b2):
    symp = symp.astype(jnp.int32)
    idx2 = symp.reshape(-1, RPC)
    lab2 = label.astype(jnp.int32).reshape(-1, LROW)
    emb_sum, emb_dise = _sc_gather(idx2, lab2, symp_table, dise_table)
    return _mlp(emb_sum, emb_dise, symp, W1,
                b1.reshape(1, D), W2.reshape(1, D), b2.reshape(1, 1))


# speedup vs baseline: 1.0073x; 1.0073x over previous
import jax
import jax.numpy as jnp
from jax import lax
from jax.experimental import pallas as pl
from jax.experimental.pallas import tpu as pltpu
from jax.experimental.pallas import tpu_sc as plsc

B = 16384
D = 64
HIST = 50
NC, NS = 2, 16
NW = NC * NS
BPW = B // NW
RPC = 2 * HIST
CHUNKS = BPW // 2
NBUF = 2
LROW = 128
LABROWS = BPW // LROW


def _accum(rows_v, c, acc_v):
    for b in range(2):
        for d in range(4):
            t = rows_v[b * HIST, pl.ds(d * 16, 16)]
            for j in range(1, HIST):
                t = t + rows_v[b * HIST + j, pl.ds(d * 16, 16)]
            acc_v[2 * c + b, pl.ds(d * 16, 16)] = t


def _sc_body(idx_hbm, lab_hbm, symp_tab, dise_tab, sum_out, dise_out,
             idx_v, lab_v, rows_v, db0_v, db1_v, acc_v,
             sem0, sem1, dsem0, dsem1):
    wid = lax.axis_index("s") * NC + lax.axis_index("c")
    pltpu.sync_copy(idx_hbm.at[pl.ds(wid * CHUNKS, CHUNKS)], idx_v)

    bufsems = (sem0, sem1)

    def gather(c, k):
        return pltpu.async_copy(
            symp_tab.at[idx_v.at[c]], rows_v.at[k], bufsems[k])

    def wait(c, k):
        pltpu.make_async_copy(
            symp_tab.at[idx_v.at[c]], rows_v.at[k], bufsems[k]).wait()

    for k in range(NBUF):
        gather(k, k)

    def quad_body(q, carry):
        c0 = q * NBUF
        for k in range(NBUF):
            wait(c0 + k, k)
            _accum(rows_v.at[k], c0 + k, acc_v)

            @pl.when(q < CHUNKS // NBUF - 1)
            def _():
                gather(c0 + k + NBUF, k)

        return carry

    lax.fori_loop(0, CHUNKS // NBUF, quad_body, 0)
    pltpu.sync_copy(acc_v, sum_out.at[pl.ds(wid * BPW, BPW)])

    pltpu.sync_copy(lab_hbm.at[pl.ds(wid * LABROWS, LABROWS)], lab_v)
    pltpu.async_copy(dise_tab.at[lab_v.at[0]], db0_v, dsem0)
    pltpu.async_copy(dise_tab.at[lab_v.at[1]], db1_v, dsem1)
    for r in range(LABROWS):
        buf = db0_v if r % 2 == 0 else db1_v
        sem = dsem0 if r % 2 == 0 else dsem1
        pltpu.make_async_copy(dise_tab.at[lab_v.at[r]], buf, sem).wait()
        pltpu.sync_copy(buf, dise_out.at[pl.ds(wid * BPW + r * LROW, LROW)])
        if r + 2 < LABROWS:
            pltpu.async_copy(dise_tab.at[lab_v.at[r + 2]], buf, sem)


_sc_gather = pl.kernel(
    _sc_body,
    out_type=(jax.ShapeDtypeStruct((B, D), jnp.float32),
              jax.ShapeDtypeStruct((B, D), jnp.float32)),
    mesh=plsc.VectorSubcoreMesh(core_axis_name="c", subcore_axis_name="s"),
    scratch_types=[
        pltpu.VMEM((CHUNKS, RPC), jnp.int32),
        pltpu.VMEM((LABROWS, LROW), jnp.int32),
        pltpu.VMEM((NBUF, RPC, D), jnp.float32),
        pltpu.VMEM((LROW, D), jnp.float32),
        pltpu.VMEM((LROW, D), jnp.float32),
        pltpu.VMEM((BPW, D), jnp.float32),
        pltpu.SemaphoreType.DMA,
        pltpu.SemaphoreType.DMA,
        pltpu.SemaphoreType.DMA,
        pltpu.SemaphoreType.DMA,
    ],
    compiler_params=pltpu.CompilerParams(use_tc_tiling_on_sc=False),
)


BLK = 256


def _mlp_body(sum_ref, dise_ref, symp_ref, w1_ref, b1_ref, w2t_ref, b2_ref,
              out_ref):
    s = symp_ref[...]
    cnt = jnp.sum((s != 0).astype(jnp.float32), axis=1, keepdims=True)
    w = 1.0 / (cnt + 1e-8)
    w = jnp.where(w >= 1e8, 0.0, w)
    w2 = jnp.reshape(w, (BLK, 2))
    es = sum_ref[...]
    ds_ = dise_ref[...]
    w1 = w1_ref[...]
    w2t = w2t_ref[...]
    b1 = b1_ref[...]
    b2 = b2_ref[...]
    outs = []
    for half in range(2):
        u = jnp.maximum(es[:, half * D:(half + 1) * D] * w2[:, half:half + 1],
                        0.0)
        dd = jnp.maximum(ds_[:, half * D:(half + 1) * D], 0.0)
        h = (jnp.dot(u, w1[:D], preferred_element_type=jnp.float32)
             + jnp.dot(dd, w1[D:], preferred_element_type=jnp.float32)
             + b1)
        h = jnp.maximum(h, 0.0)
        outs.append(jnp.sum(h * w2t, axis=1, keepdims=True) + b2)
    out_ref[...] = jnp.concatenate(outs, axis=1)


def _mlp(emb2, dise2, symp, W1, b1r, W2t, b2r):
    hist = symp.shape[1]
    return pl.pallas_call(
        _mlp_body,
        grid=(B // (2 * BLK),),
        in_specs=[
            pl.BlockSpec((BLK, 2 * D), lambda i: (i, 0)),
            pl.BlockSpec((BLK, 2 * D), lambda i: (i, 0)),
            pl.BlockSpec((2 * BLK, hist), lambda i: (i, 0)),
            pl.BlockSpec((2 * D, D), lambda i: (0, 0)),
            pl.BlockSpec((1, D), lambda i: (0, 0)),
            pl.BlockSpec((1, D), lambda i: (0, 0)),
            pl.BlockSpec((1, 1), lambda i: (0, 0)),
        ],
        out_specs=pl.BlockSpec((BLK, 2), lambda i: (i, 0)),
        out_shape=jax.ShapeDtypeStruct((B // 2, 2), jnp.float32),
    )(emb2, dise2, symp, W1, b1r, W2t, b2r)


def kernel(symp, label, symp_table, dise_table, W1, b1, W2, b2):
    symp = symp.astype(jnp.int32)
    idx2 = symp.reshape(-1, RPC)
    lab2 = label.astype(jnp.int32).reshape(-1, LROW)
    emb_sum, emb_dise = _sc_gather(idx2, lab2, symp_table, dise_table)
    emb2 = emb_sum.reshape(B // 2, 2 * D)
    dise2 = emb_dise.reshape(B // 2, 2 * D)
    out2 = _mlp(emb2, dise2, symp, W1,
                b1.reshape(1, D), W2.reshape(1, D), b2.reshape(1, 1))
    return out2.reshape(B, 1)
